# baseline (speedup 1.0000x reference)
import jax
import jax.numpy as jnp
from jax.experimental import pallas as pl
from jax.experimental.pallas import tpu as pltpu

Z_DIM = 1024
HIDDEN = 2048
W = 64
RANK = 8
N_TOK = 2048
WR = W * RANK

BN = 512
NB = N_TOK // BN


def _body(idx_ref, z_ref, hpre_ref, opre_ref, a1_ref, b1_ref, a2_ref,
          b2_ref, out_ref, a1s, a2s):
    @pl.when(pl.program_id(0) == 0)
    def _prep():
        a1s[...] = a1_ref[...].astype(jnp.bfloat16).T
        a2s[...] = a2_ref[...].astype(jnp.bfloat16).T

    idx = idx_ref[0]
    col_expert = jax.lax.broadcasted_iota(jnp.int32, (BN, WR), 1) >> 3
    mask = col_expert == idx

    zb = z_ref[...].astype(jnp.bfloat16)
    t1 = jnp.dot(zb, a1s[...], preferred_element_type=jnp.float32)
    u1 = jnp.where(mask, t1, 0.0).astype(jnp.bfloat16)
    d1 = jnp.dot(u1, b1_ref[...], preferred_element_type=jnp.float32)

    hv = hpre_ref[...] + d1
    h = (hv * jax.lax.logistic(hv)).astype(jnp.bfloat16)

    t2 = jnp.dot(h, a2s[...], preferred_element_type=jnp.float32)
    u2 = jnp.where(mask, t2, 0.0).astype(jnp.bfloat16)
    d2 = jnp.dot(u2, b2_ref[...], preferred_element_type=jnp.float32)

    out_ref[...] = opre_ref[...] + d2


def kernel(z, a_idx, h_pre, out_pre, a1, b1, a2, b2):
    idx3 = a_idx.astype(jnp.int32).reshape(NB, BN, 1)
    a1r = a1.reshape(WR, Z_DIM)
    a2r = a2.reshape(WR, HIDDEN)
    b1m = b1.swapaxes(1, 2).reshape(WR, HIDDEN).astype(jnp.bfloat16)
    b2m = b2.swapaxes(1, 2).reshape(WR, Z_DIM).astype(jnp.bfloat16)

    const = lambda shape: pl.BlockSpec(shape, lambda i: (0,) * len(shape))
    return pl.pallas_call(
        _body,
        grid=(NB,),
        in_specs=[
            pl.BlockSpec((1, BN, 1), lambda i: (i, 0, 0)),
            pl.BlockSpec((BN, Z_DIM), lambda i: (i, 0)),
            pl.BlockSpec((BN, HIDDEN), lambda i: (i, 0)),
            pl.BlockSpec((BN, Z_DIM), lambda i: (i, 0)),
            const((WR, Z_DIM)),
            const((WR, HIDDEN)),
            const((WR, HIDDEN)),
            const((WR, Z_DIM)),
        ],
        out_specs=pl.BlockSpec((BN, Z_DIM), lambda i: (i, 0)),
        out_shape=jax.ShapeDtypeStruct((N_TOK, Z_DIM), jnp.float32),
        compiler_params=pltpu.CompilerParams(
            allow_input_fusion=[False, False, False, False,
                                False, True, False, True]),
        scratch_shapes=[
            pltpu.VMEM((Z_DIM, WR), jnp.bfloat16),
            pltpu.VMEM((HIDDEN, WR), jnp.bfloat16),
        ],
    )(idx3, z, h_pre, out_pre, a1r, b1m, a2r, b2m)


# speedup vs baseline: 1.1525x; 1.1525x over previous
import jax
import jax.numpy as jnp
from jax.experimental import pallas as pl
from jax.experimental.pallas import tpu as pltpu

Z_DIM = 1024
HIDDEN = 2048
W = 64
RANK = 8
N_TOK = 2048
WR = W * RANK

BN = 512
NB = N_TOK // BN


def _body(idx_ref, z_ref, hpre_ref, opre_ref, a1_ref, b1_ref, a2_ref,
          b2_ref, out_ref, a1s, a2s, b1s, b2s):
    @pl.when(pl.program_id(0) == 0)
    def _prep():
        a1s[...] = a1_ref[...].astype(jnp.bfloat16).T
        a2s[...] = a2_ref[...].astype(jnp.bfloat16).T
        b1s[...] = b1_ref[...].astype(jnp.bfloat16)
        b2s[...] = b2_ref[...].astype(jnp.bfloat16)

    idx = idx_ref[0]
    col_expert = jax.lax.broadcasted_iota(jnp.int32, (BN, WR), 1) >> 3
    mask = col_expert == idx

    zb = z_ref[...].astype(jnp.bfloat16)
    t1 = jnp.dot(zb, a1s[...], preferred_element_type=jnp.float32)
    u1 = jnp.where(mask, t1, 0.0).astype(jnp.bfloat16)
    d1 = jnp.dot(u1, b1s[...], preferred_element_type=jnp.float32)

    hv = hpre_ref[...] + d1
    h = (hv * jax.lax.logistic(hv)).astype(jnp.bfloat16)

    t2 = jnp.dot(h, a2s[...], preferred_element_type=jnp.float32)
    u2 = jnp.where(mask, t2, 0.0).astype(jnp.bfloat16)
    d2 = jnp.dot(u2, b2s[...], preferred_element_type=jnp.float32)

    out_ref[...] = opre_ref[...] + d2


def kernel(z, a_idx, h_pre, out_pre, a1, b1, a2, b2):
    idx3 = a_idx.astype(jnp.int32).reshape(NB, BN, 1)
    a1r = a1.reshape(WR, Z_DIM)
    a2r = a2.reshape(WR, HIDDEN)
    b1m = b1.swapaxes(1, 2).reshape(WR, HIDDEN)
    b2m = b2.swapaxes(1, 2).reshape(WR, Z_DIM)

    const = lambda shape: pl.BlockSpec(shape, lambda i: (0,) * len(shape))
    return pl.pallas_call(
        _body,
        grid=(NB,),
        in_specs=[
            pl.BlockSpec((1, BN, 1), lambda i: (i, 0, 0)),
            pl.BlockSpec((BN, Z_DIM), lambda i: (i, 0)),
            pl.BlockSpec((BN, HIDDEN), lambda i: (i, 0)),
            pl.BlockSpec((BN, Z_DIM), lambda i: (i, 0)),
            const((WR, Z_DIM)),
            const((WR, HIDDEN)),
            const((WR, HIDDEN)),
            const((WR, Z_DIM)),
        ],
        out_specs=pl.BlockSpec((BN, Z_DIM), lambda i: (i, 0)),
        out_shape=jax.ShapeDtypeStruct((N_TOK, Z_DIM), jnp.float32),
        compiler_params=pltpu.CompilerParams(
            allow_input_fusion=[False, False, False, False,
                                False, True, False, True]),
        scratch_shapes=[
            pltpu.VMEM((Z_DIM, WR), jnp.bfloat16),
            pltpu.VMEM((HIDDEN, WR), jnp.bfloat16),
            pltpu.VMEM((WR, HIDDEN), jnp.bfloat16),
            pltpu.VMEM((WR, Z_DIM), jnp.bfloat16),
        ],
    )(idx3, z, h_pre, out_pre, a1r, b1m, a2r, b2m)
